# Initial kernel scaffold; baseline (speedup 1.0000x reference)
import jax
import jax.numpy as jnp
from jax.experimental import pallas as pl


def kernel(frames):
    raise NotImplementedError("write your pallas kernel here")


# speedup vs baseline: 1.2479x; 1.2479x over previous
import numpy as np
import jax
import jax.numpy as jnp
from jax.experimental import pallas as pl
from jax.experimental.pallas import tpu as pltpu

ALPHA = 4


def _slow_indices(T: int):
    return [int(v) for v in np.linspace(0, T - 1, T // ALPHA).astype(np.int64)]


def _copy_body(idx_ref, src_ref, dst_ref):
    dst_ref[...] = src_ref[...]


def kernel(frames):
    C, T, H, W = frames.shape
    idx = _slow_indices(T)
    S = len(idx)
    idx_arr = jnp.asarray(idx, dtype=jnp.int32)

    grid_spec = pltpu.PrefetchScalarGridSpec(
        num_scalar_prefetch=1,
        grid=(C, S),
        in_specs=[
            pl.BlockSpec((1, 1, H, W), lambda c, s, idx_ref: (c, idx_ref[s], 0, 0)),
        ],
        out_specs=pl.BlockSpec((1, 1, H, W), lambda c, s, idx_ref: (c, s, 0, 0)),
    )

    slow = pl.pallas_call(
        _copy_body,
        grid_spec=grid_spec,
        out_shape=jax.ShapeDtypeStruct((C, S, H, W), frames.dtype),
    )(idx_arr, frames)

    return (slow, frames)
